# baseline (speedup 1.0000x reference)
import functools

import jax
import jax.numpy as jnp
from jax import lax
from jax.experimental import pallas as pl
from jax.experimental.pallas import tpu as pltpu
from jax.experimental.pallas import tpu_sc as plsc

MAX_M = 16384
HIDDEN = 2048
NWORKERS = 32
ROWS_PER_W = MAX_M // NWORKERS
CHUNK = 16
NCH = ROWS_PER_W // CHUNK
DEPTH = 3


def _sc_body(in_hbm, pass_hbm, splits_hbm, out_hbm, bufs, msmem, sem_r, sem_w, sem_x):
    pltpu.sync_copy(splits_hbm, msmem)
    m = msmem[pl.ds(0, 1)][0]
    wid = lax.axis_index("s") * 2 + lax.axis_index("c")
    base = wid * ROWS_PER_W

    def start_read(j):
        lo = base + j * CHUNK
        buf = bufs.at[j % DEPTH]
        sem = sem_r.at[j % DEPTH]

        @pl.when(m > lo)
        def _():
            pltpu.make_async_copy(in_hbm.at[pl.ds(lo, CHUNK)], buf, sem).start()

        @pl.when(m <= lo)
        def _():
            pltpu.make_async_copy(pass_hbm.at[pl.ds(lo, CHUNK)], buf, sem).start()

    def wait_read(j):
        lo = base + j * CHUNK
        pltpu.make_async_copy(
            in_hbm.at[pl.ds(lo, CHUNK)], bufs.at[j % DEPTH], sem_r.at[j % DEPTH]
        ).wait()

    def start_write(j):
        lo = base + j * CHUNK
        pltpu.make_async_copy(
            bufs.at[j % DEPTH], out_hbm.at[pl.ds(lo, CHUNK)], sem_w.at[j % DEPTH]
        ).start()

    def wait_write(j):
        lo = base + j * CHUNK
        pltpu.make_async_copy(
            bufs.at[j % DEPTH], out_hbm.at[pl.ds(lo, CHUNK)], sem_w.at[j % DEPTH]
        ).wait()

    for j in range(NCH):
        if j >= DEPTH:
            wait_write(j - DEPTH)
        start_read(j)
        if j >= 1:
            wait_read(j - 1)
            start_write(j - 1)
    wait_read(NCH - 1)
    start_write(NCH - 1)
    for j in range(max(0, NCH - DEPTH), NCH):
        wait_write(j)

    jc_lo = (m // CHUNK) * CHUNK
    mine = jnp.logical_and(jc_lo >= base, jc_lo < base + ROWS_PER_W)
    mine = jnp.logical_and(mine, m % CHUNK != 0)
    rowbuf = bufs.at[0, pl.ds(0, 1)]
    for r in range(CHUNK):
        row = jc_lo + r

        @pl.when(jnp.logical_and(mine, jnp.logical_and(row >= m, row < MAX_M)))
        def _():
            pltpu.make_async_copy(pass_hbm.at[pl.ds(row, 1)], rowbuf, sem_x).start()
            pltpu.make_async_copy(pass_hbm.at[pl.ds(row, 1)], rowbuf, sem_x).wait()
            pltpu.make_async_copy(rowbuf, out_hbm.at[pl.ds(row, 1)], sem_x).start()
            pltpu.make_async_copy(rowbuf, out_hbm.at[pl.ds(row, 1)], sem_x).wait()


_sc_call = functools.partial(
    pl.kernel,
    mesh=plsc.VectorSubcoreMesh(core_axis_name="c", subcore_axis_name="s"),
    out_type=jax.ShapeDtypeStruct((MAX_M, HIDDEN), jnp.float32),
    scratch_types=[
        pltpu.VMEM((DEPTH, CHUNK, HIDDEN), jnp.float32),
        pltpu.VMEM((1,), jnp.int32),
        pltpu.SemaphoreType.DMA((DEPTH,)),
        pltpu.SemaphoreType.DMA((DEPTH,)),
        pltpu.SemaphoreType.DMA,
    ],
)(_sc_body)


def kernel(input, output, input_splits, output_splits, num_sm):
    del input_splits, num_sm
    return _sc_call(input, output, output_splits)


# speedup vs baseline: 1.0156x; 1.0156x over previous
import functools

import jax
import jax.numpy as jnp
from jax import lax
from jax.experimental import pallas as pl
from jax.experimental.pallas import tpu as pltpu
from jax.experimental.pallas import tpu_sc as plsc

MAX_M = 16384
HIDDEN = 2048
NWORKERS = 32
ROWS_PER_W = MAX_M // NWORKERS
CHUNK = 8
NCH = ROWS_PER_W // CHUNK
DEPTH = 7


def _sc_body(in_hbm, pass_hbm, splits_hbm, out_hbm, bufs, msmem, sem_r, sem_w, sem_x):
    pltpu.sync_copy(splits_hbm, msmem)
    m = msmem[pl.ds(0, 1)][0]
    wid = lax.axis_index("s") * 2 + lax.axis_index("c")
    base = wid * ROWS_PER_W

    def start_read(j):
        lo = base + j * CHUNK
        buf = bufs.at[j % DEPTH]
        sem = sem_r.at[j % DEPTH]

        @pl.when(m > lo)
        def _():
            pltpu.make_async_copy(in_hbm.at[pl.ds(lo, CHUNK)], buf, sem).start()

        @pl.when(m <= lo)
        def _():
            pltpu.make_async_copy(pass_hbm.at[pl.ds(lo, CHUNK)], buf, sem).start()

    def wait_read(j):
        lo = base + j * CHUNK
        pltpu.make_async_copy(
            in_hbm.at[pl.ds(lo, CHUNK)], bufs.at[j % DEPTH], sem_r.at[j % DEPTH]
        ).wait()

    def start_write(j):
        lo = base + j * CHUNK
        pltpu.make_async_copy(
            bufs.at[j % DEPTH], out_hbm.at[pl.ds(lo, CHUNK)], sem_w.at[j % DEPTH]
        ).start()

    def wait_write(j):
        lo = base + j * CHUNK
        pltpu.make_async_copy(
            bufs.at[j % DEPTH], out_hbm.at[pl.ds(lo, CHUNK)], sem_w.at[j % DEPTH]
        ).wait()

    for j in range(NCH):
        if j >= DEPTH:
            wait_write(j - DEPTH)
        start_read(j)
        if j >= 1:
            wait_read(j - 1)
            start_write(j - 1)
    wait_read(NCH - 1)
    start_write(NCH - 1)
    for j in range(max(0, NCH - DEPTH), NCH):
        wait_write(j)

    jc_lo = (m // CHUNK) * CHUNK
    mine = jnp.logical_and(jc_lo >= base, jc_lo < base + ROWS_PER_W)
    mine = jnp.logical_and(mine, m % CHUNK != 0)
    rowbuf = bufs.at[0, pl.ds(0, 1)]
    for r in range(CHUNK):
        row = jc_lo + r

        @pl.when(jnp.logical_and(mine, jnp.logical_and(row >= m, row < MAX_M)))
        def _():
            pltpu.make_async_copy(pass_hbm.at[pl.ds(row, 1)], rowbuf, sem_x).start()
            pltpu.make_async_copy(pass_hbm.at[pl.ds(row, 1)], rowbuf, sem_x).wait()
            pltpu.make_async_copy(rowbuf, out_hbm.at[pl.ds(row, 1)], sem_x).start()
            pltpu.make_async_copy(rowbuf, out_hbm.at[pl.ds(row, 1)], sem_x).wait()


_sc_call = functools.partial(
    pl.kernel,
    mesh=plsc.VectorSubcoreMesh(core_axis_name="c", subcore_axis_name="s"),
    out_type=jax.ShapeDtypeStruct((MAX_M, HIDDEN), jnp.float32),
    scratch_types=[
        pltpu.VMEM((DEPTH, CHUNK, HIDDEN), jnp.float32),
        pltpu.VMEM((1,), jnp.int32),
        pltpu.SemaphoreType.DMA((DEPTH,)),
        pltpu.SemaphoreType.DMA((DEPTH,)),
        pltpu.SemaphoreType.DMA,
    ],
)(_sc_body)


def kernel(input, output, input_splits, output_splits, num_sm):
    del input_splits, num_sm
    return _sc_call(input, output, output_splits)


# speedup vs baseline: 1.0205x; 1.0048x over previous
import functools

import jax
import jax.numpy as jnp
from jax import lax
from jax.experimental import pallas as pl
from jax.experimental.pallas import tpu as pltpu
from jax.experimental.pallas import tpu_sc as plsc

MAX_M = 16384
HIDDEN = 2048
NWORKERS = 32
ROWS_PER_W = MAX_M // NWORKERS
CHUNK = 8
NCH = ROWS_PER_W // CHUNK
DEPTH = 7


def _sc_body(in_hbm, pass_hbm, splits_hbm, out_hbm, bufs, msmem, sem_r, sem_w, sem_x):
    pltpu.sync_copy(splits_hbm, msmem)
    m = msmem[pl.ds(0, 1)][0]
    wid = lax.axis_index("c") * 16 + lax.axis_index("s")
    base = wid * ROWS_PER_W

    def start_read(j):
        lo = base + j * CHUNK
        buf = bufs.at[j % DEPTH]
        sem = sem_r.at[j % DEPTH]

        @pl.when(m > lo)
        def _():
            pltpu.make_async_copy(in_hbm.at[pl.ds(lo, CHUNK)], buf, sem).start()

        @pl.when(m <= lo)
        def _():
            pltpu.make_async_copy(pass_hbm.at[pl.ds(lo, CHUNK)], buf, sem).start()

    def wait_read(j):
        lo = base + j * CHUNK
        pltpu.make_async_copy(
            in_hbm.at[pl.ds(lo, CHUNK)], bufs.at[j % DEPTH], sem_r.at[j % DEPTH]
        ).wait()

    def start_write(j):
        lo = base + j * CHUNK
        pltpu.make_async_copy(
            bufs.at[j % DEPTH], out_hbm.at[pl.ds(lo, CHUNK)], sem_w.at[j % DEPTH]
        ).start()

    def wait_write(j):
        lo = base + j * CHUNK
        pltpu.make_async_copy(
            bufs.at[j % DEPTH], out_hbm.at[pl.ds(lo, CHUNK)], sem_w.at[j % DEPTH]
        ).wait()

    for j in range(NCH):
        if j >= DEPTH:
            wait_write(j - DEPTH)
        start_read(j)
        if j >= 1:
            wait_read(j - 1)
            start_write(j - 1)
    wait_read(NCH - 1)
    start_write(NCH - 1)
    for j in range(max(0, NCH - DEPTH), NCH):
        wait_write(j)

    jc_lo = (m // CHUNK) * CHUNK
    mine = jnp.logical_and(jc_lo >= base, jc_lo < base + ROWS_PER_W)
    mine = jnp.logical_and(mine, m % CHUNK != 0)
    rowbuf = bufs.at[0, pl.ds(0, 1)]
    for r in range(CHUNK):
        row = jc_lo + r

        @pl.when(jnp.logical_and(mine, jnp.logical_and(row >= m, row < MAX_M)))
        def _():
            pltpu.make_async_copy(pass_hbm.at[pl.ds(row, 1)], rowbuf, sem_x).start()
            pltpu.make_async_copy(pass_hbm.at[pl.ds(row, 1)], rowbuf, sem_x).wait()
            pltpu.make_async_copy(rowbuf, out_hbm.at[pl.ds(row, 1)], sem_x).start()
            pltpu.make_async_copy(rowbuf, out_hbm.at[pl.ds(row, 1)], sem_x).wait()


_sc_call = functools.partial(
    pl.kernel,
    mesh=plsc.VectorSubcoreMesh(core_axis_name="c", subcore_axis_name="s"),
    out_type=jax.ShapeDtypeStruct((MAX_M, HIDDEN), jnp.float32),
    scratch_types=[
        pltpu.VMEM((DEPTH, CHUNK, HIDDEN), jnp.float32),
        pltpu.VMEM((1,), jnp.int32),
        pltpu.SemaphoreType.DMA((DEPTH,)),
        pltpu.SemaphoreType.DMA((DEPTH,)),
        pltpu.SemaphoreType.DMA,
    ],
)(_sc_body)


def kernel(input, output, input_splits, output_splits, num_sm):
    del input_splits, num_sm
    return _sc_call(input, output, output_splits)
